# baseline (speedup 1.0000x reference)
import functools

import jax
import jax.numpy as jnp
from jax import lax
from jax.experimental import pallas as pl
from jax.experimental.pallas import tpu as pltpu
from jax.experimental.pallas import tpu_sc as plsc

NC = 2
NS = 16
NW = NC * NS
EPS = 1e-5
BM = 80
DCH = 2000
NPAD = 10240
NB_ROWS = 3


def _mmprep_body(x_ref, w_ref, dg_ref, hp_ref, dinv_ref):
    dinv = lax.rsqrt(dg_ref[0] + dg_ref[1] + 1.0)
    h = jnp.dot(x_ref[...], w_ref[...], preferred_element_type=jnp.float32)
    hp_ref[...] = h * dinv
    dinv_ref[...] = dinv


def _mmprep(x, w, deg2):
    n, _ = x.shape
    m = w.shape[1]
    return pl.pallas_call(
        _mmprep_body,
        out_shape=(jax.ShapeDtypeStruct((n, m), jnp.float32),
                   jax.ShapeDtypeStruct((n, 1), jnp.float32)),
    )(x, w, deg2)


def _combine_body(with_mm, p_ref, hp_ref, dinv_ref, b_ref, g_ref, be_ref,
                  w_ref, o_ref):
    dinv = dinv_ref[...]
    a = (p_ref[0] + p_ref[1] + hp_ref[...]) * dinv + b_ref[...]
    mean = jnp.mean(a, axis=0, keepdims=True)
    var = jnp.mean((a - mean) * (a - mean), axis=0, keepdims=True)
    xh = (a - mean) * lax.rsqrt(var + EPS)
    y = jnp.maximum(xh * g_ref[...] + be_ref[...], 0.0)
    if with_mm:
        y = jnp.dot(y, w_ref[...],
                    preferred_element_type=jnp.float32) * dinv
    o_ref[...] = y


def _combine(p, hp, dinv_col, b, g, be, w, with_mm):
    n, d = hp.shape
    m = w.shape[1] if with_mm else d
    return pl.pallas_call(
        functools.partial(_combine_body, with_mm),
        out_shape=jax.ShapeDtypeStruct((n, m), jnp.float32),
    )(p, hp, dinv_col, b.reshape(1, d), g.reshape(1, d), be.reshape(1, d), w)


def _deg_body(e, d, col_h, ew_h, deg_h,
              hist, colv0, colv1, ewv0, ewv1, idxp,
              deg_s, dsem0, dsem1, psem):
    c = lax.axis_index("c")
    s = lax.axis_index("s")
    wid = c * NS + s
    epw = e // NW
    iot = lax.iota(jnp.int32, 16)
    zero16 = jnp.zeros((16,), jnp.float32)
    ndr = NPAD // d

    def z1(i, cr):
        hist[i // 8, pl.ds((i % 8) * 16, 16)] = zero16
        return cr

    lax.fori_loop(0, NPAD // 16, z1, 0)
    zseg = ndr // NS
    pltpu.sync_copy(hist.at[pl.ds(0, zseg)],
                    deg_s.at[pl.ds(s * zseg, zseg)])
    for j in range(ndr // 16):
        idxp[pl.ds(j * 16, 16)] = j * 16 + iot
    plsc.subcore_barrier()

    nch = epw // DCH
    cbs = (colv0, colv1)
    ebs = (ewv0, ewv1)
    dsm = (dsem0, dsem1)
    base0 = wid * epw
    pltpu.async_copy(col_h.at[pl.ds(base0, DCH)], colv0, dsem0)
    pltpu.async_copy(ew_h.at[pl.ds(base0, DCH)], ewv0, dsem0)
    for ch in range(nch):
        b = ch % 2
        if ch + 1 < nch:
            nbase = base0 + (ch + 1) * DCH
            pltpu.async_copy(col_h.at[pl.ds(nbase, DCH)],
                             cbs[1 - b], dsm[1 - b])
            pltpu.async_copy(ew_h.at[pl.ds(nbase, DCH)],
                             ebs[1 - b], dsm[1 - b])
        pltpu.make_async_copy(col_h.at[pl.ds(0, DCH)], cbs[b],
                              dsm[b]).wait()
        pltpu.make_async_copy(ew_h.at[pl.ds(0, DCH)], ebs[b],
                              dsm[b]).wait()

        def dstep(k, cr):
            sl = pl.ds(k * 16, 16)
            cv = cbs[b][sl]
            w = ebs[b][sl]
            hi = cv >> 7
            lo = cv & (d - 1)
            for g in range(16):
                plsc.addupdate_scatter(hist, [hi, lo], w, mask=iot == g)
            return cr

        lax.fori_loop(0, DCH // 16, dstep, 0)

    pltpu.async_copy(hist, deg_s.at[idxp], psem, add=True)
    pltpu.make_async_copy(hist, deg_s.at[idxp], psem).wait()
    plsc.subcore_barrier()

    @pl.when(s == 0)
    def _():
        pltpu.sync_copy(deg_s, deg_h.at[c])


def _sc_deg(col, ew, d):
    e = col.shape[0]
    ndr = NPAD // d
    mesh = plsc.VectorSubcoreMesh(core_axis_name="c", subcore_axis_name="s")
    fn = pl.kernel(
        functools.partial(_deg_body, e, d),
        out_type=jax.ShapeDtypeStruct((NC, ndr, d), jnp.float32),
        mesh=mesh,
        scratch_types=[
            pltpu.VMEM((ndr, d), jnp.float32),
            pltpu.VMEM((DCH,), jnp.int32),
            pltpu.VMEM((DCH,), jnp.int32),
            pltpu.VMEM((DCH,), jnp.float32),
            pltpu.VMEM((DCH,), jnp.float32),
            pltpu.VMEM((ndr,), jnp.int32),
            pltpu.VMEM_SHARED((ndr, d), jnp.float32),
            pltpu.SemaphoreType.DMA,
            pltpu.SemaphoreType.DMA,
            pltpu.SemaphoreType.DMA,
        ],
        compiler_params=pltpu.CompilerParams(needs_layout_passes=False,
                                             use_tc_tiling_on_sc=False),
    )
    return fn(col, ew)


def _msg_body(n, e, d,
              meta_h, h_h, part_h,
              meta0, meta1, meta2,
              scidx0, scidx1, scidx2,
              ridx0, ridx1, ridx2,
              rows0, rows1, rows2,
              out_s,
              msem0, msem1, msem2,
              gsem0, gsem1, gsem2,
              ssem0, ssem1, ssem2):
    c = lax.axis_index("c")
    s = lax.axis_index("s")
    wid = c * NS + s
    epw = e // NW
    nb = epw // BM
    b0 = wid * nb
    zero16 = jnp.zeros((16,), jnp.float32)
    zrows = n // NS

    def z1(i, cr):
        rows0[i // 8, pl.ds((i % 8) * 16, 16)] = zero16
        return cr

    lax.fori_loop(0, BM * d // 16, z1, 0)
    nfull = zrows // BM
    for q in range(nfull):
        pltpu.sync_copy(rows0.at[pl.ds(0, BM)],
                        out_s.at[pl.ds(s * zrows + q * BM, BM)])
    rem = zrows - nfull * BM
    if rem:
        pltpu.sync_copy(rows0.at[pl.ds(0, rem)],
                        out_s.at[pl.ds(s * zrows + nfull * BM, rem)])
    plsc.subcore_barrier()

    bufs = ((meta0, scidx0, ridx0, rows0, msem0, gsem0, ssem0),
            (meta1, scidx1, ridx1, rows1, msem1, gsem1, ssem1),
            (meta2, scidx2, ridx2, rows2, msem2, gsem2, ssem2))

    def fill(meta, off, dst):
        for j in range(BM // 16):
            sl = pl.ds(j * 16, 16)
            dst[sl] = meta[pl.ds(off + j * 16, 16)]

    for j in range(NB_ROWS):
        pltpu.async_copy(meta_h.at[pl.ds((b0 + j) * 3 * BM, 3 * BM)],
                         bufs[j][0], bufs[j][4])
    for j in range(2):
        meta, scidx, ridx, rows, msem, gsem, ssem = bufs[j]
        pltpu.make_async_copy(meta_h.at[pl.ds(0, 3 * BM)], meta,
                              msem).wait()
        fill(meta, 0, ridx)
        pltpu.async_copy(h_h.at[ridx], rows, gsem)

    def step(i, cur, nx2):
        meta, scidx, ridx, rows, msem, gsem, ssem = cur
        nmeta, nscidx, nridx, nrows, nmsem, ngsem, nssem = nx2
        pltpu.make_async_copy(h_h.at[ridx], rows, gsem).wait()
        fill(meta, BM, scidx)

        zi16 = jnp.zeros((16,), jnp.int32)

        def scale1(ei, cr):
            fv = lax.bitcast_convert_type(
                plsc.load_gather(meta, [zi16 + (2 * BM + ei)]), jnp.float32)
            for q in range(d // 16):
                sl2 = (ei, pl.ds(q * 16, 16))
                rows[sl2] = rows[sl2] * fv
            return cr

        lax.fori_loop(0, BM, scale1, 0)
        pltpu.async_copy(rows, out_s.at[scidx], ssem, add=True)

        @pl.when(i + NB_ROWS < nb)
        def _():
            pltpu.async_copy(
                meta_h.at[pl.ds((b0 + i + NB_ROWS) * 3 * BM, 3 * BM)],
                meta, msem)

        @pl.when(i + 2 < nb)
        def _():
            @pl.when(i > 0)
            def _():
                pltpu.make_async_copy(nrows, out_s.at[nscidx], nssem).wait()

            pltpu.make_async_copy(meta_h.at[pl.ds(0, 3 * BM)], nmeta,
                                  nmsem).wait()
            fill(nmeta, 0, nridx)
            pltpu.async_copy(h_h.at[nridx], nrows, ngsem)

    def mbody(i, cr):
        for k in range(NB_ROWS):
            @pl.when(i % NB_ROWS == k)
            def _():
                step(i, bufs[k], bufs[(k + 2) % NB_ROWS])

        return cr

    lax.fori_loop(0, nb, mbody, 0)
    for j in ((nb - 2) % NB_ROWS, (nb - 1) % NB_ROWS):
        meta, scidx, ridx, rows, msem, gsem, ssem = bufs[j]
        pltpu.make_async_copy(rows, out_s.at[scidx], ssem).wait()
    plsc.subcore_barrier()

    @pl.when(s == 0)
    def _():
        pltpu.sync_copy(out_s, part_h.at[c])


def _sc_msg(meta, hp):
    n, d = hp.shape
    e = meta.shape[0] // 3
    mesh = plsc.VectorSubcoreMesh(core_axis_name="c", subcore_axis_name="s")
    scratch = (
        [pltpu.VMEM((3 * BM,), jnp.int32) for _ in range(NB_ROWS)]
        + [pltpu.VMEM((BM,), jnp.int32) for _ in range(NB_ROWS)]
        + [pltpu.VMEM((BM,), jnp.int32) for _ in range(NB_ROWS)]
        + [pltpu.VMEM((BM, d), jnp.float32) for _ in range(NB_ROWS)]
        + [pltpu.VMEM_SHARED((n, d), jnp.float32)]
        + [pltpu.SemaphoreType.DMA for _ in range(3 * NB_ROWS)]
    )
    fn = pl.kernel(
        functools.partial(_msg_body, n, e, d),
        out_type=jax.ShapeDtypeStruct((NC, n, d), jnp.float32),
        mesh=mesh,
        scratch_types=scratch,
        compiler_params=pltpu.CompilerParams(needs_layout_passes=False,
                                             use_tc_tiling_on_sc=False),
    )
    return fn(meta, hp)


def kernel(x, edge_index, edge_weight, W1, b1, g1, be1, W2, b2, g2, be2):
    row = edge_index[0].astype(jnp.int32)
    col = edge_index[1].astype(jnp.int32)
    ew = edge_weight.astype(jnp.float32)
    e = col.shape[0]
    n, d = x.shape[0], W1.shape[1]
    epw = e // NW
    nb = (epw + BM - 1) // BM
    pad = nb * BM - epw
    ewb = lax.bitcast_convert_type(ew, jnp.int32)
    parts = []
    for arr in (row, col, ewb):
        a = jnp.pad(arr.reshape(NW, epw), ((0, 0), (0, pad)))
        parts.append(a.reshape(NW, nb, BM))
    meta = jnp.stack(parts, axis=2).reshape(-1)

    deg = _sc_deg(col, ew, d)
    deg2 = deg.reshape(NC, -1)[:, :n].reshape(NC, n, 1)
    h1p, dinv_col = _mmprep(x, W1, deg2)
    p1 = _sc_msg(meta, h1p)
    h2p = _combine(p1, h1p, dinv_col, b1, g1, be1, W2, True)
    p2 = _sc_msg(meta, h2p)
    out = _combine(p2, h2p, dinv_col, b2, g2, be2, W2, False)
    return out


# speedup vs baseline: 1.1147x; 1.1147x over previous
import functools

import jax
import jax.numpy as jnp
from jax import lax
from jax.experimental import pallas as pl
from jax.experimental.pallas import tpu as pltpu
from jax.experimental.pallas import tpu_sc as plsc

NC = 2
NS = 16
NW = NC * NS
EPS = 1e-5
BM = 80
DCH = 2000
NPAD = 10240
NB_ROWS = 3


def _mmprep_body(x_ref, w_ref, dg_ref, hp_ref, dinv_ref):
    dinv = lax.rsqrt(dg_ref[0] + dg_ref[1] + 1.0)
    h = jnp.dot(x_ref[...], w_ref[...], preferred_element_type=jnp.float32)
    hp_ref[...] = h * dinv
    dinv_ref[...] = dinv


def _mmprep(x, w, deg2):
    n, _ = x.shape
    m = w.shape[1]
    return pl.pallas_call(
        _mmprep_body,
        out_shape=(jax.ShapeDtypeStruct((n, m), jnp.float32),
                   jax.ShapeDtypeStruct((n, 1), jnp.float32)),
    )(x, w, deg2)


def _combine_body(with_mm, p_ref, hp_ref, dinv_ref, b_ref, g_ref, be_ref,
                  w_ref, o_ref):
    dinv = dinv_ref[...]
    a = (p_ref[0] + p_ref[1] + hp_ref[...]) * dinv + b_ref[...]
    mean = jnp.mean(a, axis=0, keepdims=True)
    var = jnp.mean((a - mean) * (a - mean), axis=0, keepdims=True)
    xh = (a - mean) * lax.rsqrt(var + EPS)
    y = jnp.maximum(xh * g_ref[...] + be_ref[...], 0.0)
    if with_mm:
        y = jnp.dot(y, w_ref[...],
                    preferred_element_type=jnp.float32) * dinv
    o_ref[...] = y


def _combine(p, hp, dinv_col, b, g, be, w, with_mm):
    n, d = hp.shape
    m = w.shape[1] if with_mm else d
    return pl.pallas_call(
        functools.partial(_combine_body, with_mm),
        out_shape=jax.ShapeDtypeStruct((n, m), jnp.float32),
    )(p, hp, dinv_col, b.reshape(1, d), g.reshape(1, d), be.reshape(1, d), w)


def _deg_body(e, d, col_h, ew_h, deg_h,
              hist, colv0, colv1, ewv0, ewv1, idxp,
              deg_s, dsem0, dsem1, psem):
    c = lax.axis_index("c")
    s = lax.axis_index("s")
    wid = c * NS + s
    epw = e // NW
    iot = lax.iota(jnp.int32, 16)
    zero16 = jnp.zeros((16,), jnp.float32)
    ndr = NPAD // d

    def z1(i, cr):
        hist[i // 8, pl.ds((i % 8) * 16, 16)] = zero16
        return cr

    lax.fori_loop(0, NPAD // 16, z1, 0)
    zseg = ndr // NS
    pltpu.sync_copy(hist.at[pl.ds(0, zseg)],
                    deg_s.at[pl.ds(s * zseg, zseg)])
    for j in range(ndr // 16):
        idxp[pl.ds(j * 16, 16)] = j * 16 + iot
    plsc.subcore_barrier()

    nch = epw // DCH
    cbs = (colv0, colv1)
    ebs = (ewv0, ewv1)
    dsm = (dsem0, dsem1)
    base0 = wid * epw
    pltpu.async_copy(col_h.at[pl.ds(base0, DCH)], colv0, dsem0)
    pltpu.async_copy(ew_h.at[pl.ds(base0, DCH)], ewv0, dsem0)
    for ch in range(nch):
        b = ch % 2
        if ch + 1 < nch:
            nbase = base0 + (ch + 1) * DCH
            pltpu.async_copy(col_h.at[pl.ds(nbase, DCH)],
                             cbs[1 - b], dsm[1 - b])
            pltpu.async_copy(ew_h.at[pl.ds(nbase, DCH)],
                             ebs[1 - b], dsm[1 - b])
        pltpu.make_async_copy(col_h.at[pl.ds(0, DCH)], cbs[b],
                              dsm[b]).wait()
        pltpu.make_async_copy(ew_h.at[pl.ds(0, DCH)], ebs[b],
                              dsm[b]).wait()

        def dstep(k, cr):
            sl = pl.ds(k * 16, 16)
            cv = cbs[b][sl]
            w = ebs[b][sl]
            hi = cv >> 7
            lo = cv & (d - 1)
            for g in range(16):
                plsc.addupdate_scatter(hist, [hi, lo], w, mask=iot == g)
            return cr

        lax.fori_loop(0, DCH // 16, dstep, 0)

    pltpu.async_copy(hist, deg_s.at[idxp], psem, add=True)
    pltpu.make_async_copy(hist, deg_s.at[idxp], psem).wait()
    plsc.subcore_barrier()

    @pl.when(s == 0)
    def _():
        pltpu.sync_copy(deg_s, deg_h.at[c])


def _sc_deg(col, ew, d):
    e = col.shape[0]
    ndr = NPAD // d
    mesh = plsc.VectorSubcoreMesh(core_axis_name="c", subcore_axis_name="s")
    fn = pl.kernel(
        functools.partial(_deg_body, e, d),
        out_type=jax.ShapeDtypeStruct((NC, ndr, d), jnp.float32),
        mesh=mesh,
        scratch_types=[
            pltpu.VMEM((ndr, d), jnp.float32),
            pltpu.VMEM((DCH,), jnp.int32),
            pltpu.VMEM((DCH,), jnp.int32),
            pltpu.VMEM((DCH,), jnp.float32),
            pltpu.VMEM((DCH,), jnp.float32),
            pltpu.VMEM((ndr,), jnp.int32),
            pltpu.VMEM_SHARED((ndr, d), jnp.float32),
            pltpu.SemaphoreType.DMA,
            pltpu.SemaphoreType.DMA,
            pltpu.SemaphoreType.DMA,
        ],
        compiler_params=pltpu.CompilerParams(needs_layout_passes=False,
                                             use_tc_tiling_on_sc=False),
    )
    return fn(col, ew)


def _msg_body(n, e, d,
              meta_h, h_h, part_h,
              meta0, meta1, meta2,
              scidx0, scidx1, scidx2,
              ridx0, ridx1, ridx2,
              rows0, rows1, rows2,
              out_s,
              msem0, msem1, msem2,
              gsem0, gsem1, gsem2,
              ssem0, ssem1, ssem2):
    c = lax.axis_index("c")
    s = lax.axis_index("s")
    wid = c * NS + s
    epw = e // NW
    nb = epw // BM
    b0 = wid * nb
    zero16 = jnp.zeros((16,), jnp.float32)
    zrows = n // NS

    def z1(i, cr):
        rows0[i // 8, pl.ds((i % 8) * 16, 16)] = zero16
        return cr

    lax.fori_loop(0, BM * d // 16, z1, 0)
    nfull = zrows // BM
    for q in range(nfull):
        pltpu.sync_copy(rows0.at[pl.ds(0, BM)],
                        out_s.at[pl.ds(s * zrows + q * BM, BM)])
    rem = zrows - nfull * BM
    if rem:
        pltpu.sync_copy(rows0.at[pl.ds(0, rem)],
                        out_s.at[pl.ds(s * zrows + nfull * BM, rem)])
    plsc.subcore_barrier()

    bufs = ((meta0, scidx0, ridx0, rows0, msem0, gsem0, ssem0),
            (meta1, scidx1, ridx1, rows1, msem1, gsem1, ssem1),
            (meta2, scidx2, ridx2, rows2, msem2, gsem2, ssem2))

    def fill(meta, off, dst):
        for j in range(BM // 16):
            sl = pl.ds(j * 16, 16)
            dst[sl] = meta[pl.ds(off + j * 16, 16)]

    for j in range(NB_ROWS):
        pltpu.async_copy(meta_h.at[pl.ds((b0 + j) * 3 * BM, 3 * BM)],
                         bufs[j][0], bufs[j][4])
    for j in range(2):
        meta, scidx, ridx, rows, msem, gsem, ssem = bufs[j]
        pltpu.make_async_copy(meta_h.at[pl.ds(0, 3 * BM)], meta,
                              msem).wait()
        fill(meta, 0, ridx)
        pltpu.async_copy(h_h.at[ridx], rows, gsem)

    def step(i, cur, nx2):
        meta, scidx, ridx, rows, msem, gsem, ssem = cur
        nmeta, nscidx, nridx, nrows, nmsem, ngsem, nssem = nx2
        pltpu.make_async_copy(h_h.at[ridx], rows, gsem).wait()
        fill(meta, BM, scidx)

        def scale16(j, cr):
            ev = lax.bitcast_convert_type(
                meta[pl.ds(2 * BM + j * 16, 16)], jnp.float32)
            for k in range(16):
                f = ev[k]
                for q in range(d // 16):
                    sl2 = (j * 16 + k, pl.ds(q * 16, 16))
                    rows[sl2] = rows[sl2] * f
            return cr

        lax.fori_loop(0, BM // 16, scale16, 0)
        pltpu.async_copy(rows, out_s.at[scidx], ssem, add=True)

        @pl.when(i + NB_ROWS < nb)
        def _():
            pltpu.async_copy(
                meta_h.at[pl.ds((b0 + i + NB_ROWS) * 3 * BM, 3 * BM)],
                meta, msem)

        @pl.when(i + 2 < nb)
        def _():
            @pl.when(i > 0)
            def _():
                pltpu.make_async_copy(nrows, out_s.at[nscidx], nssem).wait()

            pltpu.make_async_copy(meta_h.at[pl.ds(0, 3 * BM)], nmeta,
                                  nmsem).wait()
            fill(nmeta, 0, nridx)
            pltpu.async_copy(h_h.at[nridx], nrows, ngsem)

    def mbody(i, cr):
        for k in range(NB_ROWS):
            @pl.when(i % NB_ROWS == k)
            def _():
                step(i, bufs[k], bufs[(k + 2) % NB_ROWS])

        return cr

    lax.fori_loop(0, nb, mbody, 0)
    for j in ((nb - 2) % NB_ROWS, (nb - 1) % NB_ROWS):
        meta, scidx, ridx, rows, msem, gsem, ssem = bufs[j]
        pltpu.make_async_copy(rows, out_s.at[scidx], ssem).wait()
    plsc.subcore_barrier()

    @pl.when(s == 0)
    def _():
        pltpu.sync_copy(out_s, part_h.at[c])


def _sc_msg(meta, hp):
    n, d = hp.shape
    e = meta.shape[0] // 3
    mesh = plsc.VectorSubcoreMesh(core_axis_name="c", subcore_axis_name="s")
    scratch = (
        [pltpu.VMEM((3 * BM,), jnp.int32) for _ in range(NB_ROWS)]
        + [pltpu.VMEM((BM,), jnp.int32) for _ in range(NB_ROWS)]
        + [pltpu.VMEM((BM,), jnp.int32) for _ in range(NB_ROWS)]
        + [pltpu.VMEM((BM, d), jnp.float32) for _ in range(NB_ROWS)]
        + [pltpu.VMEM_SHARED((n, d), jnp.float32)]
        + [pltpu.SemaphoreType.DMA for _ in range(3 * NB_ROWS)]
    )
    fn = pl.kernel(
        functools.partial(_msg_body, n, e, d),
        out_type=jax.ShapeDtypeStruct((NC, n, d), jnp.float32),
        mesh=mesh,
        scratch_types=scratch,
        compiler_params=pltpu.CompilerParams(needs_layout_passes=False,
                                             use_tc_tiling_on_sc=False),
    )
    return fn(meta, hp)


def kernel(x, edge_index, edge_weight, W1, b1, g1, be1, W2, b2, g2, be2):
    row = edge_index[0].astype(jnp.int32)
    col = edge_index[1].astype(jnp.int32)
    ew = edge_weight.astype(jnp.float32)
    e = col.shape[0]
    n, d = x.shape[0], W1.shape[1]
    epw = e // NW
    nb = (epw + BM - 1) // BM
    pad = nb * BM - epw
    ewb = lax.bitcast_convert_type(ew, jnp.int32)
    parts = []
    for arr in (row, col, ewb):
        a = jnp.pad(arr.reshape(NW, epw), ((0, 0), (0, pad)))
        parts.append(a.reshape(NW, nb, BM))
    meta = jnp.stack(parts, axis=2).reshape(-1)

    deg = _sc_deg(col, ew, d)
    deg2 = deg.reshape(NC, -1)[:, :n].reshape(NC, n, 1)
    h1p, dinv_col = _mmprep(x, W1, deg2)
    p1 = _sc_msg(meta, h1p)
    h2p = _combine(p1, h1p, dinv_col, b1, g1, be1, W2, True)
    p2 = _sc_msg(meta, h2p)
    out = _combine(p2, h2p, dinv_col, b2, g2, be2, W2, False)
    return out
